# baseline (speedup 1.0000x reference)
import jax
import jax.numpy as jnp
from jax.experimental import pallas as pl
from jax.experimental.pallas import tpu as pltpu

_K = 512
_N = 32768
_ROWS = 128
_R = 64
_C = 4
_NC = _N // _C


def _i32(v):
    v &= 0xFFFFFFFF
    return v - (1 << 32) if v >= (1 << 31) else v


_BIAS = _i32(0x80008000)
_LO = 0xFFFF
_MIN16 = -(2 ** 15)


def _pk16(v):
    return pltpu.bitcast(v, jnp.int16)


def _count_pk(mask):
    m16 = mask.astype(jnp.int16)
    return jnp.sum(pltpu.bitcast(m16, jnp.int32), axis=1, keepdims=True)


def _sel_pk(ge_lo, ge_hi, a, b):
    lo = jnp.where(ge_lo, a, b) & _LO
    hi = jnp.where(ge_hi, a, b) & ~_LO
    return lo | hi


def _keys16(x):
    b = jax.lax.bitcast_convert_type(x, jnp.uint32)
    h16 = jax.lax.bitcast_convert_type((b >> 16).astype(jnp.uint16),
                                       jnp.int16)
    l16 = jax.lax.bitcast_convert_type(
        (b & jnp.uint32(_LO)).astype(jnp.uint16), jnp.int16)
    isneg = h16 < 0
    hs = jnp.where(isneg, h16 ^ jnp.int16(0x7FFF), h16)
    ls = l16 ^ jnp.where(isneg, jnp.int16(0x7FFF), jnp.int16(_MIN16))
    return hs, ls


def _topk_mask_body(x_ref, o_ref, hs_scr, ls_scr, lop_scr, st_scr):
    j = pl.program_id(1)
    k = jnp.int32(_K)

    @pl.when(j < _C)
    def _convert():
        hs, ls = _keys16(x_ref[...])
        hs_scr[j] = hs
        ls_scr[j] = ls

    @pl.when(j == _C - 1)
    def _bisect():
        p_pk = jnp.zeros((_R // 2, 1), jnp.int32)
        for jb in range(15, -1, -1):
            cand = p_pk | jnp.int32(_i32((1 << jb) | (1 << (jb + 16))))
            c16 = _pk16(cand ^ _BIAS)
            s = jnp.zeros((_R // 2, 1), jnp.int32)
            for c in range(_C):
                s = s + _count_pk(hs_scr[c] >= c16)
            c_lo = s & _LO
            c_hi = jax.lax.shift_right_logical(s, 16)
            p_pk = _sel_pk(c_lo >= k, c_hi >= k, cand, p_pk)

        ps16 = _pk16(p_pk ^ _BIAS)

        for c in range(_C):
            hs = hs_scr[c]
            lop_scr[c] = jnp.where(hs >= ps16,
                                   jnp.where(hs == ps16, ls_scr[c],
                                             jnp.int16(0x7FFF)),
                                   jnp.int16(_MIN16))

        q_pk = jnp.zeros((_R // 2, 1), jnp.int32)
        for jb in range(15, -1, -1):
            cand = q_pk | jnp.int32(_i32((1 << jb) | (1 << (jb + 16))))
            c16 = _pk16(cand ^ _BIAS)
            s = jnp.zeros((_R // 2, 1), jnp.int32)
            for c in range(_C):
                s = s + _count_pk(lop_scr[c] >= c16)
            c_lo = s & _LO
            c_hi = jax.lax.shift_right_logical(s, 16)
            q_pk = _sel_pk(c_lo >= k, c_hi >= k, cand, q_pk)

        pb_pk = p_pk ^ _BIAS
        qb_pk = q_pk ^ _BIAS
        pb_l = (pb_pk << 16) >> 16
        pb_h = pb_pk >> 16
        qb_l = (qb_pk << 16) >> 16
        qb_h = qb_pk >> 16
        th_l = jnp.where(pb_l > 0, pb_l, 0)
        th_h = jnp.where(pb_h > 0, pb_h, 0)
        tl_l = jnp.where(pb_l >= 0, qb_l, _MIN16)
        tl_h = jnp.where(pb_h >= 0, qb_h, _MIN16)
        st_scr[0] = (th_l & _LO) | (th_h << 16)
        st_scr[1] = (tl_l & _LO) | (tl_h << 16)

    @pl.when(j >= _C)
    def _emit():
        c = j - _C
        th = _pk16(st_scr[0])
        tl = _pk16(st_scr[1])
        hs = hs_scr[c]
        ls = ls_scr[c]
        keep = (hs > th) | ((hs == th) & (ls >= tl))
        o_ref[...] = jnp.where(keep, x_ref[...], 0.0)


@jax.jit
def kernel(x):
    return pl.pallas_call(
        _topk_mask_body,
        grid=(_ROWS // _R, 2 * _C),
        in_specs=[pl.BlockSpec((_R, _NC),
                               lambda i, j: (i, jax.lax.rem(j, _C)))],
        out_specs=pl.BlockSpec((_R, _NC),
                               lambda i, j: (i, jnp.maximum(j - _C, 0))),
        out_shape=jax.ShapeDtypeStruct((_ROWS, _N), jnp.float32),
        scratch_shapes=[
            pltpu.VMEM((_C, _R, _NC), jnp.int16),
            pltpu.VMEM((_C, _R, _NC), jnp.int16),
            pltpu.VMEM((_C, _R, _NC), jnp.int16),
            pltpu.VMEM((2, _R // 2, 1), jnp.int32),
        ],
    )(x)


# speedup vs baseline: 1.1330x; 1.1330x over previous
import jax
import jax.numpy as jnp
from jax.experimental import pallas as pl
from jax.experimental.pallas import tpu as pltpu

_K = 512
_N = 32768
_ROWS = 128
_R = 64


def _i32(v):
    v &= 0xFFFFFFFF
    return v - (1 << 32) if v >= (1 << 31) else v


_BIAS = _i32(0x80008000)
_LO = 0xFFFF
_MIN16 = -(2 ** 15)


def _pk16(v):
    return pltpu.bitcast(v, jnp.int16)


def _count_pk(mask):
    m16 = mask.astype(jnp.int16)
    return jnp.sum(pltpu.bitcast(m16, jnp.int32), axis=1, keepdims=True)


def _halves(s):
    return s & _LO, jax.lax.shift_right_logical(s, 16)


def _sel_pk(ge_lo, ge_hi, a, b):
    lo = jnp.where(ge_lo, a, b) & _LO
    hi = jnp.where(ge_hi, a, b) & ~_LO
    return lo | hi


def _topk_mask_body(x_ref, o_ref):
    x = x_ref[...]
    b = jax.lax.bitcast_convert_type(x, jnp.uint32)

    h16 = jax.lax.bitcast_convert_type((b >> 16).astype(jnp.uint16),
                                       jnp.int16)
    l16 = jax.lax.bitcast_convert_type(
        (b & jnp.uint32(_LO)).astype(jnp.uint16), jnp.int16)
    isneg = h16 < 0
    hs = jnp.where(isneg, h16 ^ jnp.int16(0x7FFF), h16)
    ls = l16 ^ jnp.where(isneg, jnp.int16(0x7FFF), jnp.int16(_MIN16))

    k = jnp.int32(_K)

    p_pk = jnp.zeros((_R // 2, 1), jnp.int32)
    for j in range(15, -1, -1):
        cand = p_pk | jnp.int32(_i32((1 << j) | (1 << (j + 16))))
        c_lo, c_hi = _halves(_count_pk(hs >= _pk16(cand ^ _BIAS)))
        p_pk = _sel_pk(c_lo >= k, c_hi >= k, cand, p_pk)

    ps16 = _pk16(p_pk ^ _BIAS)
    lop = jnp.where(hs >= ps16,
                    jnp.where(hs == ps16, ls, jnp.int16(0x7FFF)),
                    jnp.int16(_MIN16))

    q_pk = jnp.zeros((_R // 2, 1), jnp.int32)
    for j in range(15, -1, -1):
        cand = q_pk | jnp.int32(_i32((1 << j) | (1 << (j + 16))))
        c_lo, c_hi = _halves(_count_pk(lop >= _pk16(cand ^ _BIAS)))
        q_pk = _sel_pk(c_lo >= k, c_hi >= k, cand, q_pk)

    pb_pk = p_pk ^ _BIAS
    qb_pk = q_pk ^ _BIAS
    pb_l = (pb_pk << 16) >> 16
    pb_h = pb_pk >> 16
    qb_l = (qb_pk << 16) >> 16
    qb_h = qb_pk >> 16
    th_l = jnp.where(pb_l > 0, pb_l, 0)
    th_h = jnp.where(pb_h > 0, pb_h, 0)
    tl_l = jnp.where(pb_l >= 0, qb_l, _MIN16)
    tl_h = jnp.where(pb_h >= 0, qb_h, _MIN16)
    th = _pk16((th_l & _LO) | (th_h << 16))
    tl = _pk16((tl_l & _LO) | (tl_h << 16))
    keep = (hs > th) | ((hs == th) & (ls >= tl))
    o_ref[...] = jnp.where(keep, x, 0.0)


@jax.jit
def kernel(x):
    return pl.pallas_call(
        _topk_mask_body,
        grid=(_ROWS // _R,),
        in_specs=[pl.BlockSpec((_R, _N), lambda i: (i, 0))],
        out_specs=pl.BlockSpec((_R, _N), lambda i: (i, 0)),
        out_shape=jax.ShapeDtypeStruct((_ROWS, _N), jnp.float32),
    )(x)
